# baseline (speedup 1.0000x reference)
import jax
import jax.numpy as jnp
from jax import lax
from jax.experimental import pallas as pl
from jax.experimental.pallas import tpu as pltpu
from jax.experimental.pallas import tpu_sc as plsc

H = 32
W = 32
D = 256
B = 32
F = 2 * D
LANES = 16


def _pos_body(row_hbm, col_hbm, out_hbm, row2_v, build_v, pos_sh):
    c = lax.axis_index("c")
    s = lax.axis_index("s")

    pltpu.sync_copy(col_hbm, build_v.at[pl.ds(0, W), pl.ds(0, D)])
    pltpu.sync_copy(col_hbm, build_v.at[pl.ds(W, W), pl.ds(0, D)])

    pltpu.sync_copy(row_hbm.at[pl.ds(2 * s, 2)], row2_v)

    for r in range(2):
        vs = [row2_v[r, pl.ds(j * LANES, LANES)] for j in range(D // LANES)]

        def st(i, carry, vs=vs, r=r):
            for j in range(D // LANES):
                build_v[r * W + i, pl.ds(D + j * LANES, LANES)] = vs[j]
            return carry

        lax.fori_loop(0, W, st, 0)

    pltpu.sync_copy(build_v, pos_sh.at[pl.ds(s * 64, 64)])
    plsc.subcore_barrier()

    b = c * 16 + s
    pltpu.sync_copy(pos_sh, out_hbm.at[b])


_pos_kernel = pl.kernel(
    _pos_body,
    out_type=jax.ShapeDtypeStruct((B, H * W, F), jnp.float32),
    mesh=plsc.VectorSubcoreMesh(core_axis_name="c", subcore_axis_name="s"),
    scratch_types=[
        pltpu.VMEM((2, D), jnp.float32),
        pltpu.VMEM((64, F), jnp.float32),
        pltpu.VMEM_SHARED((H * W, F), jnp.float32),
    ],
)


def kernel(x, row_w, col_w):
    del x
    return _pos_kernel(row_w, col_w)


# speedup vs baseline: 1.2841x; 1.2841x over previous
import jax
import jax.numpy as jnp
from jax import lax
from jax.experimental import pallas as pl
from jax.experimental.pallas import tpu as pltpu
from jax.experimental.pallas import tpu_sc as plsc

H = 32
W = 32
D = 256
B = 32
F = 2 * D
LANES = 16


NSTREAM_C = 9
NB_STREAM = 2 * NSTREAM_C
NB_SP = B - NB_STREAM
NSP_C = NB_SP // 2
NHALF = 2 * NSP_C


def _pos_body(row_hbm, col_hbm, out_hbm, row2_v, build_v, pos_sh, sem):
    c = lax.axis_index("c")
    s = lax.axis_index("s")

    pltpu.sync_copy(col_hbm, build_v.at[pl.ds(0, W), pl.ds(0, D)])
    pltpu.sync_copy(col_hbm, build_v.at[pl.ds(W, W), pl.ds(0, D)])

    pltpu.sync_copy(row_hbm.at[pl.ds(2 * s, 2)], row2_v)

    for r in range(2):
        vs = [row2_v[r, pl.ds(j * LANES, LANES)] for j in range(D // LANES)]

        def st(i, carry, vs=vs, r=r):
            for j in range(D // LANES):
                build_v[r * W + i, pl.ds(D + j * LANES, LANES)] = vs[j]
            return carry

        lax.fori_loop(0, W, st, 0)

    copies = [
        pltpu.async_copy(
            build_v,
            out_hbm.at[c * NSTREAM_C + j, pl.ds(s * 64, 64)],
            sem,
        )
        for j in range(NSTREAM_C)
    ]

    pltpu.sync_copy(build_v, pos_sh.at[pl.ds(s * 64, 64)])
    plsc.subcore_barrier()

    @pl.when(s < NHALF)
    def _sp_copy():
        bsp = NB_STREAM + c * NSP_C + s // 2
        r0 = (s % 2) * 512
        pltpu.sync_copy(pos_sh.at[pl.ds(r0, 512)], out_hbm.at[bsp, pl.ds(r0, 512)])

    for cp in copies:
        cp.wait()


_pos_kernel = pl.kernel(
    _pos_body,
    out_type=jax.ShapeDtypeStruct((B, H * W, F), jnp.float32),
    mesh=plsc.VectorSubcoreMesh(core_axis_name="c", subcore_axis_name="s"),
    scratch_types=[
        pltpu.VMEM((2, D), jnp.float32),
        pltpu.VMEM((64, F), jnp.float32),
        pltpu.VMEM_SHARED((H * W, F), jnp.float32),
        pltpu.SemaphoreType.DMA,
    ],
)


def kernel(x, row_w, col_w):
    del x
    return _pos_kernel(row_w, col_w)
